# baseline (speedup 1.0000x reference)
import functools

import jax
import jax.numpy as jnp
import numpy as np
from jax import lax
from jax.experimental import pallas as pl
from jax.experimental.pallas import tpu as pltpu
from jax.experimental.pallas import tpu_sc as plsc

N = 16777216
NUM_BINS = 256
LOW_P = 0.01
HIGH_P = 0.99
EPS = 1e-6

NBIN = 65536
NTILES = 32
PER_TILE = N // NTILES
CHUNK = 16384
NCHUNK = PER_TILE // CHUNK

T_LO = float(np.float32(LOW_P) * np.float32(N - 1))
T_HI = float(np.float32(HIGH_P) * np.float32(N - 1))


def _hist_body(vals_hbm, out_hbm, buf_a, buf_b, hist, sem_a, sem_b):
    c = lax.axis_index("c")
    s = lax.axis_index("s")
    wid = s * 2 + c
    base = wid * PER_TILE

    zero16 = jnp.zeros((16,), jnp.int32)
    one16 = jnp.ones((16,), jnp.int32)
    topbit = jnp.full((16,), -2147483648, jnp.int32)

    def zbody(i, carry):
        hist[pl.ds(i * 16, 16)] = zero16
        return carry

    lax.fori_loop(0, NBIN // 16, zbody, 0)

    def copy_in(g, buf, sem):
        off = pl.multiple_of(base + g * CHUNK, 8)
        return pltpu.make_async_copy(vals_hbm.at[pl.ds(off, CHUNK)], buf, sem)

    copy_in(0, buf_a, sem_a).start()

    def process(buf):
        @plsc.parallel_loop(0, CHUNK // 16, unroll=8)
        def _(i):
            v = buf[pl.ds(i * 16, 16)]
            b = lax.bitcast_convert_type(v, jnp.int32)
            sgn = lax.shift_right_arithmetic(b, 31)
            m = lax.bitwise_or(sgn, topbit)
            key = lax.bitwise_xor(b, m)
            hi = lax.shift_right_logical(key, 16)
            plsc.addupdate_scatter(hist, [hi], one16)

    def chunk_body(g2, carry):
        for k in range(2):
            g = g2 * 2 + k
            buf, sem = (buf_a, sem_a) if k == 0 else (buf_b, sem_b)
            nbuf, nsem = (buf_b, sem_b) if k == 0 else (buf_a, sem_a)
            copy_in(g, buf, sem).wait()

            @pl.when(g + 1 < NCHUNK)
            def _():
                copy_in(g + 1, nbuf, nsem).start()

            process(buf)
        return carry

    lax.fori_loop(0, NCHUNK // 2, chunk_body, 0)
    pltpu.sync_copy(hist, out_hbm.at[wid])


@functools.cache
def _hist_call():
    return pl.kernel(
        _hist_body,
        out_type=jax.ShapeDtypeStruct((NTILES, NBIN), jnp.int32),
        mesh=plsc.VectorSubcoreMesh(core_axis_name="c", subcore_axis_name="s",
                                    num_cores=2, num_subcores=16),
        compiler_params=pltpu.CompilerParams(needs_layout_passes=False),
        scratch_types=[
            pltpu.VMEM((CHUNK,), jnp.float32),
            pltpu.VMEM((CHUNK,), jnp.float32),
            pltpu.VMEM((NBIN,), jnp.int32),
            pltpu.SemaphoreType.DMA,
            pltpu.SemaphoreType.DMA,
        ],
    )


def _quantiles_from_hist(h_ref):
    acc = jnp.zeros((512, 128), jnp.float32)
    for t in range(NTILES):
        acc = acc + h_ref[t * 512:(t + 1) * 512, :].astype(jnp.float32)

    ci = lax.broadcasted_iota(jnp.int32, (128, 128), 0)
    cj = lax.broadcasted_iota(jnp.int32, (128, 128), 1)
    ut = (ci <= cj).astype(jnp.float32)
    C = jnp.dot(acc, ut, preferred_element_type=jnp.float32,
                precision=lax.Precision.HIGHEST)

    ri = lax.broadcasted_iota(jnp.int32, (512, 512), 0)
    rj = lax.broadcasted_iota(jnp.int32, (512, 512), 1)
    lt = (rj < ri).astype(jnp.float32)
    rowpre = jnp.dot(lt, C, preferred_element_type=jnp.float32,
                     precision=lax.Precision.HIGHEST)[:, 127:128]
    cum = C + rowpre

    flat = (lax.broadcasted_iota(jnp.int32, (512, 128), 0) * 128
            + lax.broadcasted_iota(jnp.int32, (512, 128), 1)
            ).astype(jnp.float32)

    def locate(t):
        mask = cum <= t
        bstar = jnp.sum(mask.astype(jnp.float32))
        cumbelow = jnp.max(jnp.where(mask, cum, 0.0))
        cnt = jnp.sum(jnp.where(flat == bstar, acc, 0.0))
        pos = jnp.clip((t - cumbelow + 0.5) / jnp.maximum(cnt, 1.0), 0.0, 1.0)
        low16 = jnp.clip(jnp.floor(pos * 65536.0), 0.0, 65535.0)
        key = lax.bitwise_or(
            lax.shift_left(bstar.astype(jnp.int32), 16), low16.astype(jnp.int32))
        kt = lax.shift_right_arithmetic(key, 31)
        mm = lax.bitwise_or(lax.bitwise_not(kt), jnp.int32(-2147483648))
        bits = lax.bitwise_xor(key, mm)
        return lax.bitcast_convert_type(bits, jnp.float32)

    lo = locate(T_LO)
    hi = locate(T_HI)
    inv = 1.0 / (hi - lo + EPS)
    return lo, inv


BLKN = 1048576


def _mapq_body(h_ref, v_ref, o_ref, scr_ref):
    @pl.when(pl.program_id(0) == 0)
    def _():
        lo, inv = _quantiles_from_hist(h_ref)
        scr_ref[0] = lo
        scr_ref[1] = inv

    lo = scr_ref[0]
    inv = scr_ref[1]
    v = v_ref[...]
    norm = jnp.clip((v - lo) * inv, 0.0, 1.0)
    q = jnp.clip(jnp.floor(norm * 256.0), 0.0, 255.0)
    o_ref[...] = (q + 0.5) * (1.0 / 256.0)


_mapq_call = pl.pallas_call(
    _mapq_body,
    grid=(N // BLKN,),
    in_specs=[
        pl.BlockSpec((NTILES * 512, 128), lambda i: (0, 0)),
        pl.BlockSpec((BLKN,), lambda i: (i,)),
    ],
    out_specs=pl.BlockSpec((BLKN,), lambda i: (i,)),
    out_shape=jax.ShapeDtypeStruct((N,), jnp.float32),
    scratch_shapes=[pltpu.SMEM((2,), jnp.float32)],
)


def kernel(values):
    hists = _hist_call()(values)
    return _mapq_call(hists.reshape(NTILES * 512, 128), values)


# speedup vs baseline: 1.2202x; 1.2202x over previous
import functools

import jax
import jax.numpy as jnp
import numpy as np
from jax import lax
from jax.experimental import pallas as pl
from jax.experimental.pallas import tpu as pltpu
from jax.experimental.pallas import tpu_sc as plsc

N = 16777216
NUM_BINS = 256
LOW_P = 0.01
HIGH_P = 0.99
EPS = 1e-6

NBIN = 65536
NTILES = 32
PER_TILE = N // NTILES
CHUNK = 16384
NCHUNK = PER_TILE // CHUNK

T_LO = float(np.float32(LOW_P) * np.float32(N - 1))
T_HI = float(np.float32(HIGH_P) * np.float32(N - 1))


def _hist_body(vals_hbm, out_hbm, buf_a, buf_b, hist, sem_a, sem_b):
    c = lax.axis_index("c")
    s = lax.axis_index("s")
    wid = s * 2 + c
    base = wid * PER_TILE

    zero16 = jnp.zeros((16,), jnp.int32)
    one16 = jnp.ones((16,), jnp.int32)
    topbit = jnp.full((16,), -2147483648, jnp.int32)

    def zbody(r, carry):
        for u in range(8):
            hist[r, pl.ds(u * 16, 16)] = zero16
        return carry

    lax.fori_loop(0, 512, zbody, 0)

    def copy_in(g, buf, sem):
        off = pl.multiple_of(base + g * CHUNK, 8)
        return pltpu.make_async_copy(vals_hbm.at[pl.ds(off, CHUNK)], buf, sem)

    copy_in(0, buf_a, sem_a).start()

    def process(buf):
        @plsc.parallel_loop(0, CHUNK // 16, unroll=8)
        def _(i):
            v = buf[pl.ds(i * 16, 16)]
            b = lax.bitcast_convert_type(v, jnp.int32)
            sgn = lax.shift_right_arithmetic(b, 31)
            m = lax.bitwise_or(sgn, topbit)
            key = lax.bitwise_xor(b, m)
            hi = lax.shift_right_logical(key, 16)
            row = lax.shift_right_logical(hi, 7)
            col = lax.bitwise_and(hi, jnp.full((16,), 127, jnp.int32))
            plsc.addupdate_scatter(hist, [row, col], one16)

    def chunk_body(g2, carry):
        for k in range(2):
            g = g2 * 2 + k
            buf, sem = (buf_a, sem_a) if k == 0 else (buf_b, sem_b)
            nbuf, nsem = (buf_b, sem_b) if k == 0 else (buf_a, sem_a)
            copy_in(g, buf, sem).wait()

            @pl.when(g + 1 < NCHUNK)
            def _():
                copy_in(g + 1, nbuf, nsem).start()

            process(buf)
        return carry

    lax.fori_loop(0, NCHUNK // 2, chunk_body, 0)
    pltpu.sync_copy(hist, out_hbm.at[pl.ds(wid * 512, 512), :])


@functools.cache
def _hist_call():
    return pl.kernel(
        _hist_body,
        out_type=jax.ShapeDtypeStruct((NTILES * 512, 128), jnp.int32),
        mesh=plsc.VectorSubcoreMesh(core_axis_name="c", subcore_axis_name="s",
                                    num_cores=2, num_subcores=16),
        compiler_params=pltpu.CompilerParams(needs_layout_passes=False),
        scratch_types=[
            pltpu.VMEM((CHUNK,), jnp.float32),
            pltpu.VMEM((CHUNK,), jnp.float32),
            pltpu.VMEM((512, 128), jnp.int32),
            pltpu.SemaphoreType.DMA,
            pltpu.SemaphoreType.DMA,
        ],
    )


def _quantiles_from_hist(h_ref):
    acc = jnp.zeros((512, 128), jnp.float32)
    for t in range(NTILES):
        acc = acc + h_ref[t * 512:(t + 1) * 512, :].astype(jnp.float32)

    ci = lax.broadcasted_iota(jnp.int32, (128, 128), 0)
    cj = lax.broadcasted_iota(jnp.int32, (128, 128), 1)
    ut = (ci <= cj).astype(jnp.float32)
    C = jnp.dot(acc, ut, preferred_element_type=jnp.float32,
                precision=lax.Precision.HIGHEST)

    ri = lax.broadcasted_iota(jnp.int32, (512, 512), 0)
    rj = lax.broadcasted_iota(jnp.int32, (512, 512), 1)
    lt = (rj < ri).astype(jnp.float32)
    rowpre = jnp.dot(lt, C, preferred_element_type=jnp.float32,
                     precision=lax.Precision.HIGHEST)[:, 127:128]
    cum = C + rowpre

    flat = (lax.broadcasted_iota(jnp.int32, (512, 128), 0) * 128
            + lax.broadcasted_iota(jnp.int32, (512, 128), 1)
            ).astype(jnp.float32)

    def locate(t):
        mask = cum <= t
        bstar = jnp.sum(mask.astype(jnp.float32))
        cumbelow = jnp.max(jnp.where(mask, cum, 0.0))
        cnt = jnp.sum(jnp.where(flat == bstar, acc, 0.0))
        pos = jnp.clip((t - cumbelow + 0.5) / jnp.maximum(cnt, 1.0), 0.0, 1.0)
        low16 = jnp.clip(jnp.floor(pos * 65536.0), 0.0, 65535.0)
        key = lax.bitwise_or(
            lax.shift_left(bstar.astype(jnp.int32), 16), low16.astype(jnp.int32))
        kt = lax.shift_right_arithmetic(key, 31)
        mm = lax.bitwise_or(lax.bitwise_not(kt), jnp.int32(-2147483648))
        bits = lax.bitwise_xor(key, mm)
        return lax.bitcast_convert_type(bits, jnp.float32)

    lo = locate(T_LO)
    hi = locate(T_HI)
    inv = 1.0 / (hi - lo + EPS)
    return lo, inv


BLKN = 2097152


def _mapq_body(h_ref, v_ref, o_ref, scr_ref):
    @pl.when(pl.program_id(0) == 0)
    def _():
        lo, inv = _quantiles_from_hist(h_ref)
        scr_ref[0] = lo
        scr_ref[1] = inv

    lo = scr_ref[0]
    inv = scr_ref[1]
    v = v_ref[...]
    norm = jnp.clip((v - lo) * inv, 0.0, 1.0)
    q = jnp.clip(jnp.floor(norm * 256.0), 0.0, 255.0)
    o_ref[...] = (q + 0.5) * (1.0 / 256.0)


_mapq_call = pl.pallas_call(
    _mapq_body,
    grid=(N // BLKN,),
    in_specs=[
        pl.BlockSpec((NTILES * 512, 128), lambda i: (0, 0)),
        pl.BlockSpec((BLKN,), lambda i: (i,)),
    ],
    out_specs=pl.BlockSpec((BLKN,), lambda i: (i,)),
    out_shape=jax.ShapeDtypeStruct((N,), jnp.float32),
    scratch_shapes=[pltpu.SMEM((2,), jnp.float32)],
)


def kernel(values):
    hists = _hist_call()(values)
    return _mapq_call(hists, values)
